# baseline (speedup 1.0000x reference)
import functools
import numpy as np
import jax
import jax.numpy as jnp
from jax import lax
from jax.experimental import pallas as pl
from jax.experimental.pallas import tpu as pltpu
from jax.experimental.pallas import tpu_sc as plsc

_N = 8192
_B = 128
_K = 16
_Z = 1.0 / (1.0 - float(np.exp(-1.0)))

_R = 8
_T = 128

_NW = 32
_IPW = _B * _K // _NW


def _rank_lse_kernel(scores_ref, rel_ref, lse_ref, idx_ref):
    pid = pl.program_id(0)

    row_i = jax.lax.broadcasted_iota(jnp.int32, (_K, _T), 0)
    tcol_j = jax.lax.broadcasted_iota(jnp.int32, (_K, _T), 1)
    target_col = row_i + (_T - _K)
    sel = tcol_j == target_col

    acc = jnp.zeros((1, 1), jnp.float32)
    for row in range(_R):
        s = scores_ref[row:row + 1, :]
        r_head = rel_ref[row:row + 1, : _N - _T]
        r_tail = rel_ref[row:row + 1, _N - _T:]

        mx = jnp.max(s, axis=1, keepdims=True)
        acc = acc + mx + jnp.log(jnp.sum(jnp.exp(s - mx), axis=1, keepdims=True))

        v = jnp.sum(jnp.where(sel, r_tail, 0.0), axis=1, keepdims=True)

        head_cnt = jnp.sum(jnp.where(r_head <= v, 1.0, 0.0),
                           axis=1, keepdims=True)
        hit = (r_tail < v) | ((r_tail == v) & (tcol_j < target_col))
        tail_cnt = jnp.sum(jnp.where(hit, 1.0, 0.0), axis=1, keepdims=True)
        ranks = head_cnt + tail_cnt

        gbase = (pid * _R + row) * _N
        gidx = ranks.astype(jnp.int32) + gbase
        idx_ref[row * _K:(row + 1) * _K, :] = gidx

    @pl.when(pid == 0)
    def _():
        lse_ref[...] = jnp.zeros_like(lse_ref)

    lse_ref[...] += acc


def _sc_gather_dot(scores_hbm, idx_hbm, out_hbm, idx_v, vals_v, part_v, sem):
    wid = lax.axis_index("s") * 2 + lax.axis_index("c")
    base = wid * _IPW
    pltpu.sync_copy(idx_hbm.at[pl.ds(base, _IPW)], idx_v)
    pltpu.async_copy(scores_hbm.at[idx_v], vals_v, sem).wait()

    lane = lax.iota(jnp.int32, 16).astype(jnp.float32)
    w16 = jnp.exp(lane - float(_K - 1))

    vsum = jnp.zeros((16,), jnp.float32)
    for c in range(_IPW // 16):
        vsum = vsum + vals_v[pl.ds(c * 16, 16)]
    part_v[...] = vsum * w16
    pltpu.sync_copy(part_v, out_hbm.at[wid])


def _combine_kernel(lse_ref, parts_ref, out_ref):
    total_dot = jnp.sum(parts_ref[...], axis=0, keepdims=True)
    total_dot = jnp.sum(total_dot, axis=1, keepdims=True)
    out_ref[...] = (lse_ref[...] - total_dot * (1.0 / _Z)) * (1.0 / _B)


def kernel(scores, relevance):
    lse_sum, gidx = pl.pallas_call(
        _rank_lse_kernel,
        grid=(_B // _R,),
        in_specs=[
            pl.BlockSpec((_R, _N), lambda i: (i, 0)),
            pl.BlockSpec((_R, _N), lambda i: (i, 0)),
        ],
        out_specs=[
            pl.BlockSpec((1, 1), lambda i: (0, 0)),
            pl.BlockSpec((_R * _K, 1), lambda i: (i, 0)),
        ],
        out_shape=[
            jax.ShapeDtypeStruct((1, 1), jnp.float32),
            jax.ShapeDtypeStruct((_B * _K, 1), jnp.int32),
        ],
    )(scores, relevance)

    sc_gather = functools.partial(
        pl.kernel,
        mesh=plsc.VectorSubcoreMesh(core_axis_name="c", subcore_axis_name="s"),
        out_type=jax.ShapeDtypeStruct((_NW, 16), jnp.float32),
        scratch_types=[
            pltpu.VMEM((_IPW,), jnp.int32),
            pltpu.VMEM((_IPW,), jnp.float32),
            pltpu.VMEM((16,), jnp.float32),
            pltpu.SemaphoreType.DMA,
        ],
    )(_sc_gather_dot)
    parts = sc_gather(scores.reshape(_B * _N), gidx.reshape(_B * _K))

    out = pl.pallas_call(
        _combine_kernel,
        in_specs=[
            pl.BlockSpec((1, 1), lambda: (0, 0)),
            pl.BlockSpec((_NW, 16), lambda: (0, 0)),
        ],
        out_specs=pl.BlockSpec((1, 1), lambda: (0, 0)),
        out_shape=jax.ShapeDtypeStruct((1, 1), jnp.float32),
    )(lse_sum, parts)
    return out[0, 0]
